# baseline (speedup 1.0000x reference)
import functools

import jax
import jax.numpy as jnp
from jax.experimental import pallas as pl
from jax.experimental.pallas import tpu as pltpu
from jax.experimental.pallas import tpu_sc as plsc

_TEMP = 0.1
_TOPK = 32
_N = 4096
_D = 256
_BLK = 256
_NW = 32
_RPW = _N // _NW
_LANES = 16
_NVREG = _N // _LANES
_NCHUNK = 32
_VPC = 8
_CANDBUF = _N + _LANES

_NEG = float("-inf")


def _sim_kernel(xb_ref, xf_ref, out_ref):
    out_ref[...] = jax.lax.dot_general(
        xb_ref[...], xf_ref[...],
        dimension_numbers=(((1,), (1,)), ((), ())),
        preferred_element_type=jnp.float32,
    )


def _adj_kernel(sim_ref, tcol_ref, trow_ref, out_ref):
    raw = sim_ref[...]
    in_row = (raw >= tcol_ref[...]).astype(jnp.float32)
    in_col = (raw >= trow_ref[...]).astype(jnp.float32)
    out_ref[...] = (raw / jnp.float32(_TEMP)) * ((in_row + in_col) * 0.5)


def _thr_sc_body(sim_hbm, thr_hbm, row_v, cand_v, thr_v, sem):
    wid = jax.lax.axis_index("s") * 2 + jax.lax.axis_index("c")
    row0 = wid * _RPW
    neg16 = jnp.full((_LANES,), _NEG, jnp.float32)

    def do_row(r, tvec):
        pltpu.async_copy(sim_hbm.at[row0 + r], row_v, sem).wait()

        def chunk_max(c, L):
            acc = neg16
            for k in range(_VPC):
                acc = jnp.maximum(acc, row_v[pl.ds(c * 128 + k * _LANES,
                                                   _LANES)])
            return jnp.minimum(L, jnp.max(acc))

        L = jax.lax.fori_loop(0, _NCHUNK, chunk_max, jnp.float32(jnp.inf))
        L16 = jnp.full((_LANES,), L, jnp.float32)

        def compact(i, cnt):
            v = row_v[pl.ds(i * _LANES, _LANES)]
            m = v >= L16
            plsc.store_compressed(cand_v.at[pl.ds(cnt, _LANES)], v, mask=m)
            return cnt + jnp.max(plsc.all_reduce_population_count(m))

        cnt = jax.lax.fori_loop(0, _NVREG, compact, jnp.int32(0))
        cand_v[pl.ds(cnt, _LANES)] = neg16
        nv = cnt // _LANES + 1

        def knock(t, _):
            def vmax(j, acc):
                return jnp.maximum(acc, cand_v[pl.ds(j * _LANES, _LANES)])
            m = jnp.max(jax.lax.fori_loop(0, nv, vmax, neg16))
            m16 = jnp.full((_LANES,), m, jnp.float32)

            def kill(j, _):
                v = cand_v[pl.ds(j * _LANES, _LANES)]
                cand_v[pl.ds(j * _LANES, _LANES)] = jnp.where(
                    v >= m16, neg16, v)
                return 0
            jax.lax.fori_loop(0, nv, kill, 0)
            return 0

        jax.lax.fori_loop(0, _TOPK - 1, knock, 0)

        def vmax(j, acc):
            return jnp.maximum(acc, cand_v[pl.ds(j * _LANES, _LANES)])
        t32 = jnp.max(jax.lax.fori_loop(0, nv, vmax, neg16))

        lane = jnp.mod(r, _LANES)
        tvec = jnp.where(jax.lax.iota(jnp.int32, _LANES) == lane, t32, tvec)

        @pl.when(lane == _LANES - 1)
        def _():
            thr_v[pl.ds((r // _LANES) * _LANES, _LANES)] = tvec
        return tvec

    jax.lax.fori_loop(0, _RPW, do_row, neg16)
    pltpu.sync_copy(thr_v, thr_hbm.at[pl.ds(row0, _RPW)])


_thr_sc = functools.partial(
    pl.kernel,
    out_type=jax.ShapeDtypeStruct((_N,), jnp.float32),
    mesh=plsc.VectorSubcoreMesh(core_axis_name="c", subcore_axis_name="s"),
    compiler_params=pltpu.CompilerParams(needs_layout_passes=False),
    scratch_types=[
        pltpu.VMEM((_N,), jnp.float32),
        pltpu.VMEM((_CANDBUF,), jnp.float32),
        pltpu.VMEM((_RPW,), jnp.float32),
        pltpu.SemaphoreType.DMA,
    ],
)(_thr_sc_body)


def kernel(x):
    nblk = _N // _BLK
    sim = pl.pallas_call(
        _sim_kernel,
        grid=(nblk,),
        in_specs=[
            pl.BlockSpec((_BLK, _D), lambda i: (i, 0)),
            pl.BlockSpec((_N, _D), lambda i: (0, 0)),
        ],
        out_specs=pl.BlockSpec((_BLK, _N), lambda i: (i, 0)),
        out_shape=jax.ShapeDtypeStruct((_N, _N), jnp.float32),
    )(x, x)

    thr = _thr_sc(sim)
    tcol = thr.reshape(_N, 1)
    trow = thr.reshape(1, _N)

    adj = pl.pallas_call(
        _adj_kernel,
        grid=(nblk,),
        in_specs=[
            pl.BlockSpec((_BLK, _N), lambda i: (i, 0)),
            pl.BlockSpec((_BLK, 1), lambda i: (i, 0)),
            pl.BlockSpec((1, _N), lambda i: (0, 0)),
        ],
        out_specs=pl.BlockSpec((_BLK, _N), lambda i: (i, 0)),
        out_shape=jax.ShapeDtypeStruct((_N, _N), jnp.float32),
    )(sim, tcol, trow)
    return adj


# speedup vs baseline: 1.4575x; 1.4575x over previous
import functools

import jax
import jax.numpy as jnp
from jax.experimental import pallas as pl
from jax.experimental.pallas import tpu as pltpu
from jax.experimental.pallas import tpu_sc as plsc

_TEMP = 0.1
_TOPK = 32
_N = 4096
_D = 256
_BLK = 256
_NW = 32
_RPW = _N // _NW
_LANES = 16
_NVREG = _N // _LANES
_CANDBUF = _N + _LANES

_NEG = float("-inf")


def _sim_kernel(xb_ref, xf_ref, sim_ref, lb_ref):
    raw = jax.lax.dot_general(
        xb_ref[...], xf_ref[...],
        dimension_numbers=(((1,), (1,)), ((), ())),
        preferred_element_type=jnp.float32,
    )
    sim_ref[...] = raw

    m = raw[:, 0:128]
    for c in range(1, 32):
        m = jnp.maximum(m, raw[:, c * 128:(c + 1) * 128])

    def knock_out(_, s):
        mx = jnp.max(s, axis=1, keepdims=True)
        return jnp.where(s >= mx, -jnp.inf, s)

    m = jax.lax.fori_loop(0, _TOPK - 1, knock_out, m)
    lb_ref[...] = jnp.max(m, axis=1, keepdims=True)


def _adj_kernel(sim_ref, tcol_ref, trow_ref, out_ref):
    raw = sim_ref[...]
    in_row = (raw >= tcol_ref[...]).astype(jnp.float32)
    in_col = (raw >= trow_ref[...]).astype(jnp.float32)
    out_ref[...] = (raw / jnp.float32(_TEMP)) * ((in_row + in_col) * 0.5)


def _u32_key(v):
    bu = plsc.bitcast(v, jnp.uint32)
    flip = jnp.where(bu >= jnp.uint32(0x80000000),
                     jnp.uint32(0xFFFFFFFF), jnp.uint32(0x80000000))
    return bu ^ flip


def _thr_sc_body(sim_hbm, lb_hbm, thr_hbm, row0_v, row1_v, cand_v, key_v,
                 lb_v, thr_v, sem0, sem1):
    wid = jax.lax.axis_index("s") * 2 + jax.lax.axis_index("c")
    base = wid * _RPW
    neg16 = jnp.full((_LANES,), _NEG, jnp.float32)

    pltpu.async_copy(lb_hbm.at[pl.ds(base, _RPW)], lb_v, sem0).wait()
    pltpu.async_copy(sim_hbm.at[base], row0_v, sem0)

    def do_row(r, tvec):
        even = jax.lax.rem(r, 2) == 0

        @pl.when(r + 1 < _RPW)
        def _():
            @pl.when(even)
            def _():
                pltpu.async_copy(sim_hbm.at[base + r + 1], row1_v, sem1)

            @pl.when(jnp.logical_not(even))
            def _():
                pltpu.async_copy(sim_hbm.at[base + r + 1], row0_v, sem0)

        def run(row_v, sem):
            pltpu.make_async_copy(sim_hbm.at[base + r], row_v, sem).wait()
            L16 = plsc.load_gather(
                lb_v, [jnp.full((_LANES,), r, jnp.int32)])
            L = jnp.max(L16)

            def compact(i, cnt):
                for u in range(4):
                    v = row_v[pl.ds((4 * i + u) * _LANES, _LANES)]
                    m = v >= L16
                    plsc.store_compressed(cand_v.at[pl.ds(cnt, _LANES)],
                                          v, mask=m)
                    cnt = cnt + plsc.all_reduce_population_count(m)[0]
                return cnt

            cnt = jax.lax.fori_loop(0, _NVREG // 4, compact, jnp.int32(0))
            cand_v[pl.ds(cnt, _LANES)] = neg16
            nv = cnt // _LANES + 1

            def to_key(j, _):
                key_v[pl.ds(j * _LANES, _LANES)] = _u32_key(
                    cand_v[pl.ds(j * _LANES, _LANES)])
                return 0

            jax.lax.fori_loop(0, nv, to_key, 0)

            blo = jax.lax.bitcast_convert_type(L, jnp.uint32)
            lflip = jnp.where(blo >= jnp.uint32(0x80000000),
                              jnp.uint32(0xFFFFFFFF), jnp.uint32(0x80000000))
            lo0 = blo ^ lflip

            def bisect(b, lohi):
                lo, hi = lohi
                mid = lo + ((hi - lo + jnp.uint32(1)) >> 1)
                mid16 = jnp.full((_LANES,), mid, jnp.uint32)

                def count(j, c):
                    k = key_v[pl.ds(j * _LANES, _LANES)]
                    return c + plsc.all_reduce_population_count(
                        k >= mid16)[0]

                c = jax.lax.fori_loop(0, nv, count, jnp.int32(0))
                good = c >= _TOPK
                return (jnp.where(good, mid, lo),
                        jnp.where(good, hi, mid - jnp.uint32(1)))

            lo, _hi = jax.lax.fori_loop(
                0, 32, bisect, (lo0, jnp.uint32(0xFF7FFFFF)))

            tflip = jnp.where(lo >= jnp.uint32(0x80000000),
                              jnp.uint32(0x80000000), jnp.uint32(0xFFFFFFFF))
            return jax.lax.bitcast_convert_type(lo ^ tflip, jnp.float32)

        t32_even = jax.lax.cond(
            even, lambda: run(row0_v, sem0), lambda: run(row1_v, sem1))

        lane = jnp.mod(r, _LANES)
        tvec = jnp.where(jax.lax.iota(jnp.int32, _LANES) == lane,
                         t32_even, tvec)

        @pl.when(lane == _LANES - 1)
        def _():
            thr_v[pl.ds((r // _LANES) * _LANES, _LANES)] = tvec
        return tvec

    jax.lax.fori_loop(0, _RPW, do_row, neg16)
    pltpu.sync_copy(thr_v, thr_hbm.at[pl.ds(base, _RPW)])


_thr_sc = functools.partial(
    pl.kernel,
    out_type=jax.ShapeDtypeStruct((_N,), jnp.float32),
    mesh=plsc.VectorSubcoreMesh(core_axis_name="c", subcore_axis_name="s"),
    compiler_params=pltpu.CompilerParams(needs_layout_passes=False),
    scratch_types=[
        pltpu.VMEM((_N,), jnp.float32),
        pltpu.VMEM((_N,), jnp.float32),
        pltpu.VMEM((_CANDBUF,), jnp.float32),
        pltpu.VMEM((_CANDBUF,), jnp.uint32),
        pltpu.VMEM((_RPW,), jnp.float32),
        pltpu.VMEM((_RPW,), jnp.float32),
        pltpu.SemaphoreType.DMA,
        pltpu.SemaphoreType.DMA,
    ],
)(_thr_sc_body)


def kernel(x):
    nblk = _N // _BLK
    sim, lb = pl.pallas_call(
        _sim_kernel,
        grid=(nblk,),
        in_specs=[
            pl.BlockSpec((_BLK, _D), lambda i: (i, 0)),
            pl.BlockSpec((_N, _D), lambda i: (0, 0)),
        ],
        out_specs=[
            pl.BlockSpec((_BLK, _N), lambda i: (i, 0)),
            pl.BlockSpec((_BLK, 1), lambda i: (i, 0)),
        ],
        out_shape=[
            jax.ShapeDtypeStruct((_N, _N), jnp.float32),
            jax.ShapeDtypeStruct((_N, 1), jnp.float32),
        ],
    )(x, x)

    thr = _thr_sc(sim, lb.reshape(_N))
    tcol = thr.reshape(_N, 1)
    trow = thr.reshape(1, _N)

    adj = pl.pallas_call(
        _adj_kernel,
        grid=(nblk,),
        in_specs=[
            pl.BlockSpec((_BLK, _N), lambda i: (i, 0)),
            pl.BlockSpec((_BLK, 1), lambda i: (i, 0)),
            pl.BlockSpec((1, _N), lambda i: (0, 0)),
        ],
        out_specs=pl.BlockSpec((_BLK, _N), lambda i: (i, 0)),
        out_shape=jax.ShapeDtypeStruct((_N, _N), jnp.float32),
    )(sim, tcol, trow)
    return adj
